# Initial kernel scaffold; baseline (speedup 1.0000x reference)
import jax
import jax.numpy as jnp
from jax.experimental import pallas as pl


def kernel(comment, emb_table, W, b):
    raise NotImplementedError("write your pallas kernel here")


# speedup vs baseline: 36.4965x; 36.4965x over previous
import dataclasses
import functools

import jax
import jax.numpy as jnp
from jax import lax
from jax.experimental import pallas as pl
from jax.experimental.pallas import tpu as pltpu
from jax.experimental.pallas import tpu_sc as plsc

_NC = 2
_NS = 16
_LANES = 16
_NBUF = 4


def _sc_pool(comment_pad, emb_table):
    B, LP = comment_pad.shape
    L = 200
    V, D = emb_table.shape
    NW = _NC * _NS
    ROWS = B // NW
    C0 = 128
    C1 = L - C0

    mesh = plsc.VectorSubcoreMesh(core_axis_name="c", subcore_axis_name="s")
    cp = pltpu.CompilerParams(needs_layout_passes=False,
                              use_tc_tiling_on_sc=False)

    @functools.partial(
        pl.kernel,
        out_type=jax.ShapeDtypeStruct((B, D), jnp.float32),
        mesh=mesh,
        compiler_params=cp,
        scratch_types=[
            pltpu.VMEM((ROWS, LP), jnp.int32),
            pltpu.VMEM((_NBUF, L, D), jnp.float32),
            pltpu.VMEM((ROWS, D), jnp.float32),
            pltpu.SemaphoreType.DMA,
            pltpu.SemaphoreType.DMA,
            pltpu.SemaphoreType.DMA,
            pltpu.SemaphoreType.DMA,
        ],
    )
    def k(comment_hbm, table_hbm, out_hbm, idx_v, buf_v, pool_v, s0, s1, s2, s3):
        sems = (s0, s1, s2, s3)
        cid = lax.axis_index("c")
        sid = lax.axis_index("s")
        wid = sid * _NC + cid
        base = wid * ROWS

        pltpu.sync_copy(comment_hbm.at[pl.ds(base, ROWS)], idx_v)

        def fire(b, r):
            pltpu.async_copy(
                table_hbm.at[idx_v.at[r, pl.ds(0, C0)]],
                buf_v.at[b, pl.ds(0, C0)], sems[b])
            pltpu.async_copy(
                table_hbm.at[idx_v.at[r, pl.ds(C0, C1)]],
                buf_v.at[b, pl.ds(C0, C1)], sems[b])

        for b in range(_NBUF):
            fire(b, b)

        @pl.loop(0, ROWS, step=_NBUF)
        def _(r0):
            for b in range(_NBUF):
                r = r0 + b
                pltpu.make_async_copy(
                    table_hbm.at[pl.ds(0, L)], buf_v.at[b], sems[b]).wait()

                cnt = jnp.zeros((_LANES,), jnp.int32)
                ones = jnp.ones((_LANES,), jnp.int32)
                zeros = jnp.zeros((_LANES,), jnp.int32)
                for kk in range(LP // _LANES):
                    v = idx_v[r, pl.ds(kk * _LANES, _LANES)]
                    cnt = cnt + jnp.where(v != 0, ones, zeros)
                length = jnp.sum(cnt)

                neg = jnp.full((_LANES,), -1e9, jnp.float32)
                n8 = length // 8

                def blk_body(i, carry):
                    a0, a1 = carry
                    for j in range(8):
                        p = i * 8 + j
                        a0 = jnp.maximum(a0, buf_v[b, p, pl.ds(0, _LANES)])
                        a1 = jnp.maximum(a1, buf_v[b, p, pl.ds(_LANES, _LANES)])
                    return a0, a1

                a0, a1 = lax.fori_loop(0, n8, blk_body, (neg, neg))

                def rem_body(p, carry):
                    a0, a1 = carry
                    a0 = jnp.maximum(a0, buf_v[b, p, pl.ds(0, _LANES)])
                    a1 = jnp.maximum(a1, buf_v[b, p, pl.ds(_LANES, _LANES)])
                    return a0, a1

                a0, a1 = lax.fori_loop(n8 * 8, length, rem_body, (a0, a1))

                pool_v[r, pl.ds(0, _LANES)] = a0
                pool_v[r, pl.ds(_LANES, _LANES)] = a1

                @pl.when(r + _NBUF < ROWS)
                def _():
                    fire(b, r + _NBUF)

        pltpu.sync_copy(pool_v, out_hbm.at[pl.ds(base, ROWS)])

    return k(comment_pad, emb_table)


def _logits_mm(pooled, W, b):
    B, D = pooled.shape
    C = W.shape[1]

    def mm(x_ref, w_ref, b_ref, o_ref):
        o_ref[...] = (
            jnp.dot(x_ref[...], w_ref[...], preferred_element_type=jnp.float32)
            + b_ref[...]
        )

    return pl.pallas_call(
        mm,
        out_shape=jax.ShapeDtypeStruct((B, C), jnp.float32),
    )(pooled, W, b.reshape(1, C))


def kernel(comment, emb_table, W, b):
    comment = comment.astype(jnp.int32)
    lp = (comment.shape[1] + 15) // 16 * 16
    comment_pad = jnp.pad(comment, ((0, 0), (0, lp - comment.shape[1])))
    pooled = _sc_pool(comment_pad, emb_table)
    return _logits_mm(pooled, W, b)


# speedup vs baseline: 58.8189x; 1.6116x over previous
import dataclasses
import functools

import jax
import jax.numpy as jnp
from jax import lax
from jax.experimental import pallas as pl
from jax.experimental.pallas import tpu as pltpu
from jax.experimental.pallas import tpu_sc as plsc

_NC = 2
_NS = 16
_LANES = 16
_NBUF = 4


def _sc_pool(comment_pad, emb_table):
    B, LP = comment_pad.shape
    L = 200
    V, D = emb_table.shape
    NW = _NC * _NS
    ROWS = B // NW
    C0 = 128
    C1 = L - C0

    mesh = plsc.VectorSubcoreMesh(core_axis_name="c", subcore_axis_name="s")
    cp = pltpu.CompilerParams(needs_layout_passes=False,
                              use_tc_tiling_on_sc=False)

    @functools.partial(
        pl.kernel,
        out_type=jax.ShapeDtypeStruct((B, D), jnp.float32),
        mesh=mesh,
        compiler_params=cp,
        scratch_types=[
            pltpu.VMEM((ROWS, LP), jnp.int32),
            pltpu.VMEM((ROWS, LP), jnp.int32),
            pltpu.VMEM((_NBUF, L, D), jnp.float32),
            pltpu.VMEM((ROWS, D), jnp.float32),
            pltpu.SMEM((ROWS,), jnp.int32),
            pltpu.SemaphoreType.DMA,
            pltpu.SemaphoreType.DMA,
            pltpu.SemaphoreType.DMA,
            pltpu.SemaphoreType.DMA,
        ],
    )
    def k(comment_hbm, table_hbm, out_hbm, idx_v, idxm_v, buf_v, pool_v,
          len_sm, s0, s1, s2, s3):
        sems = (s0, s1, s2, s3)
        cid = lax.axis_index("c")
        sid = lax.axis_index("s")
        wid = sid * _NC + cid
        base = wid * ROWS

        pltpu.sync_copy(comment_hbm.at[pl.ds(base, ROWS)], idx_v)

        ones = jnp.ones((_LANES,), jnp.int32)
        zeros = jnp.zeros((_LANES,), jnp.int32)

        @pl.loop(0, ROWS)
        def _(rr):
            cnt = zeros
            for kk in range(LP // _LANES):
                v = idx_v[rr, pl.ds(kk * _LANES, _LANES)]
                cnt = cnt + jnp.where(v != 0, ones, zeros)
                g = ((v & (_KPACK - 1)) << 2) | (v >> 18)
                idxm_v[rr, pl.ds(kk * _LANES, _LANES)] = g
            len_sm[rr] = jnp.sum(cnt)

        def fire(b, r):
            pltpu.async_copy(
                table_hbm.at[idxm_v.at[r, pl.ds(0, C0)]],
                buf_v.at[b, pl.ds(0, C0)], sems[b])
            pltpu.async_copy(
                table_hbm.at[idxm_v.at[r, pl.ds(C0, C1)]],
                buf_v.at[b, pl.ds(C0, C1)], sems[b])

        for b in range(_NBUF):
            fire(b, b)

        @pl.loop(0, ROWS, step=_NBUF)
        def _(r0):
            for b in range(_NBUF):
                r = r0 + b
                pltpu.make_async_copy(
                    table_hbm.at[pl.ds(0, L)], buf_v.at[b], sems[b]).wait()

                length = len_sm[r]
                neg = jnp.full((_LANES,), -1e9, jnp.float32)
                n8 = length // 8

                def blk_body(i, carry):
                    a0, a1 = carry
                    for j in range(8):
                        p = i * 8 + j
                        a0 = jnp.maximum(a0, buf_v[b, p, pl.ds(0, _LANES)])
                        a1 = jnp.maximum(a1, buf_v[b, p, pl.ds(_LANES, _LANES)])
                    return a0, a1

                a0, a1 = lax.fori_loop(0, n8, blk_body, (neg, neg))

                def rem_body(p, carry):
                    a0, a1 = carry
                    a0 = jnp.maximum(a0, buf_v[b, p, pl.ds(0, _LANES)])
                    a1 = jnp.maximum(a1, buf_v[b, p, pl.ds(_LANES, _LANES)])
                    return a0, a1

                a0, a1 = lax.fori_loop(n8 * 8, length, rem_body, (a0, a1))

                pool_v[r, pl.ds(0, _LANES)] = a0
                pool_v[r, pl.ds(_LANES, _LANES)] = a1

                @pl.when(r + _NBUF < ROWS)
                def _():
                    fire(b, r + _NBUF)

        pltpu.sync_copy(pool_v, out_hbm.at[pl.ds(base, ROWS)])

    return k(comment_pad, emb_table)


_KPACK = 1 << 18


def _to_linear(tT):
    D, V = tT.shape
    R = 2048
    NQ = 4

    last_blk = (V - 1) // R

    def tr(x0, x1, x2, x3, o_ref):
        o_ref[...] = jnp.concatenate(
            [x0[...].T, x1[...].T, x2[...].T, x3[...].T], axis=1)

    return pl.pallas_call(
        tr,
        grid=(_KPACK // R,),
        in_specs=[
            pl.BlockSpec(
                (D, R),
                lambda i, q=q: (0, jnp.minimum(q * (_KPACK // R) + i,
                                               last_blk)))
            for q in range(NQ)
        ],
        out_specs=pl.BlockSpec((R, 4 * D), lambda i: (i, 0)),
        out_shape=jax.ShapeDtypeStruct((_KPACK, 4 * D), jnp.float32),
    )(tT, tT, tT, tT)


def _logits_mm(pooled, W, b):
    B, D = pooled.shape
    C = W.shape[1]

    def mm(x_ref, w_ref, b_ref, o_ref):
        o_ref[...] = (
            jnp.dot(x_ref[...], w_ref[...], preferred_element_type=jnp.float32)
            + b_ref[...]
        )

    return pl.pallas_call(
        mm,
        out_shape=jax.ShapeDtypeStruct((B, C), jnp.float32),
    )(pooled, W, b.reshape(1, C))


def kernel(comment, emb_table, W, b):
    comment = comment.astype(jnp.int32)
    lp = (comment.shape[1] + 15) // 16 * 16
    comment_pad = jnp.pad(comment, ((0, 0), (0, lp - comment.shape[1])))
    lin = _to_linear(emb_table.T)
    table_lin = lin.reshape(-1).reshape(4 * _KPACK, emb_table.shape[1])
    pooled = _sc_pool(comment_pad, table_lin)
    return _logits_mm(pooled, W, b)


# speedup vs baseline: 91.4188x; 1.5542x over previous
import dataclasses
import functools

import jax
import jax.numpy as jnp
from jax import lax
from jax.experimental import pallas as pl
from jax.experimental.pallas import tpu as pltpu
from jax.experimental.pallas import tpu_sc as plsc

_NC = 2
_NS = 16
_LANES = 16
_NBUF = 4


def _sc_pool(comment_pad, emb_table):
    B, LP = comment_pad.shape
    L = 200
    V, D = emb_table.shape
    NW = _NC * _NS
    ROWS = B // NW
    C0 = 128
    C1 = L - C0

    mesh = plsc.VectorSubcoreMesh(core_axis_name="c", subcore_axis_name="s")
    cp = pltpu.CompilerParams(needs_layout_passes=False,
                              use_tc_tiling_on_sc=False)

    @functools.partial(
        pl.kernel,
        out_type=jax.ShapeDtypeStruct((B, D), jnp.float32),
        mesh=mesh,
        compiler_params=cp,
        scratch_types=[
            pltpu.VMEM((ROWS, LP), jnp.int32),
            pltpu.VMEM((ROWS, LP), jnp.int32),
            pltpu.VMEM((_NBUF, L, D), jnp.float32),
            pltpu.VMEM((ROWS, D), jnp.float32),
            pltpu.SMEM((ROWS,), jnp.int32),
            pltpu.SemaphoreType.DMA,
            pltpu.SemaphoreType.DMA,
            pltpu.SemaphoreType.DMA,
            pltpu.SemaphoreType.DMA,
        ],
    )
    def k(comment_hbm, table_hbm, out_hbm, idx_v, idxm_v, buf_v, pool_v,
          len_sm, s0, s1, s2, s3):
        sems = (s0, s1, s2, s3)
        cid = lax.axis_index("c")
        sid = lax.axis_index("s")
        wid = sid * _NC + cid
        base = wid * ROWS

        pltpu.sync_copy(comment_hbm.at[pl.ds(base, ROWS)], idx_v)

        ones = jnp.ones((_LANES,), jnp.int32)
        zeros = jnp.zeros((_LANES,), jnp.int32)

        @pl.loop(0, ROWS)
        def _(rr):
            cnt = zeros
            for kk in range(LP // _LANES):
                v = idx_v[rr, pl.ds(kk * _LANES, _LANES)]
                cnt = cnt + jnp.where(v != 0, ones, zeros)
                g = ((v & (_KPACK - 1)) << 2) | (v >> 18)
                idxm_v[rr, pl.ds(kk * _LANES, _LANES)] = g
            len_sm[rr] = jnp.sum(cnt)

        def fire(b, r):
            pltpu.async_copy(
                table_hbm.at[idxm_v.at[r, pl.ds(0, C0)]],
                buf_v.at[b, pl.ds(0, C0)], sems[b])
            pltpu.async_copy(
                table_hbm.at[idxm_v.at[r, pl.ds(C0, C1)]],
                buf_v.at[b, pl.ds(C0, C1)], sems[b])

        for b in range(_NBUF):
            fire(b, b)

        @pl.loop(0, ROWS, step=_NBUF)
        def _(r0):
            for b in range(_NBUF):
                r = r0 + b
                pltpu.make_async_copy(
                    table_hbm.at[pl.ds(0, L)], buf_v.at[b], sems[b]).wait()

                length = len_sm[r]
                neg = jnp.full((_LANES,), -1e9, jnp.float32)
                n8 = length // 8

                def blk_body(i, carry):
                    a0, a1 = carry
                    for j in range(8):
                        p = i * 8 + j
                        a0 = jnp.maximum(a0, buf_v[b, p, pl.ds(0, _LANES)])
                        a1 = jnp.maximum(a1, buf_v[b, p, pl.ds(_LANES, _LANES)])
                    return a0, a1

                a0, a1 = lax.fori_loop(0, n8, blk_body, (neg, neg))

                def rem_body(p, carry):
                    a0, a1 = carry
                    a0 = jnp.maximum(a0, buf_v[b, p, pl.ds(0, _LANES)])
                    a1 = jnp.maximum(a1, buf_v[b, p, pl.ds(_LANES, _LANES)])
                    return a0, a1

                a0, a1 = lax.fori_loop(n8 * 8, length, rem_body, (a0, a1))

                pool_v[r, pl.ds(0, _LANES)] = a0
                pool_v[r, pl.ds(_LANES, _LANES)] = a1

                @pl.when(r + _NBUF < ROWS)
                def _():
                    fire(b, r + _NBUF)

        pltpu.sync_copy(pool_v, out_hbm.at[pl.ds(base, ROWS)])

    return k(comment_pad, emb_table)


_KPACK = 1 << 18


def _to_linear(tT):
    D, V = tT.shape
    R = 2048
    NQ = 4

    last_blk = (V - 1) // R

    def tr(x0, x1, x2, x3, o_ref):
        o_ref[...] = jnp.concatenate(
            [x0[...], x1[...], x2[...], x3[...]], axis=0).T

    return pl.pallas_call(
        tr,
        grid=(_KPACK // R,),
        in_specs=[
            pl.BlockSpec(
                (D, R),
                lambda i, q=q: (0, jnp.minimum(q * (_KPACK // R) + i,
                                               last_blk)))
            for q in range(NQ)
        ],
        out_specs=pl.BlockSpec((R, 4 * D), lambda i: (i, 0)),
        out_shape=jax.ShapeDtypeStruct((_KPACK, 4 * D), jnp.float32),
    )(tT, tT, tT, tT)


def _logits_mm(pooled, W, b):
    B, D = pooled.shape
    C = W.shape[1]

    def mm(x_ref, w_ref, b_ref, o_ref):
        o_ref[...] = (
            jnp.dot(x_ref[...], w_ref[...], preferred_element_type=jnp.float32)
            + b_ref[...]
        )

    return pl.pallas_call(
        mm,
        out_shape=jax.ShapeDtypeStruct((B, C), jnp.float32),
    )(pooled, W, b.reshape(1, C))


def kernel(comment, emb_table, W, b):
    comment = comment.astype(jnp.int32)
    lp = (comment.shape[1] + 15) // 16 * 16
    comment_pad = jnp.pad(comment, ((0, 0), (0, lp - comment.shape[1])))
    lin = _to_linear(emb_table.T)
    table_lin = lin.reshape(-1).reshape(4 * _KPACK, emb_table.shape[1])
    pooled = _sc_pool(comment_pad, table_lin)
    return _logits_mm(pooled, W, b)


# speedup vs baseline: 119.9322x; 1.3119x over previous
import dataclasses
import functools

import jax
import jax.numpy as jnp
from jax import lax
from jax.experimental import pallas as pl
from jax.experimental.pallas import tpu as pltpu
from jax.experimental.pallas import tpu_sc as plsc

_NC = 2
_NS = 16
_LANES = 16
_NBUF = 4


def _sc_pool(comment_pad, emb_table):
    B, LP = comment_pad.shape
    L = 200
    V, D = emb_table.shape
    NW = _NC * _NS
    ROWS = B // NW
    C0 = 128
    C1 = L - C0

    mesh = plsc.VectorSubcoreMesh(core_axis_name="c", subcore_axis_name="s")
    cp = pltpu.CompilerParams(needs_layout_passes=False,
                              use_tc_tiling_on_sc=False)

    @functools.partial(
        pl.kernel,
        out_type=jax.ShapeDtypeStruct((B, D), jnp.float32),
        mesh=mesh,
        compiler_params=cp,
        scratch_types=[
            pltpu.VMEM((ROWS, LP), jnp.int32),
            pltpu.VMEM((ROWS, LP), jnp.int32),
            pltpu.VMEM((_NBUF, L, D), jnp.float32),
            pltpu.VMEM((ROWS, D), jnp.float32),
            pltpu.SMEM((ROWS,), jnp.int32),
            pltpu.SemaphoreType.DMA,
            pltpu.SemaphoreType.DMA,
            pltpu.SemaphoreType.DMA,
            pltpu.SemaphoreType.DMA,
        ],
    )
    def k(comment_hbm, table_hbm, out_hbm, idx_v, idxm_v, buf_v, pool_v,
          len_sm, s0, s1, s2, s3):
        sems = (s0, s1, s2, s3)
        cid = lax.axis_index("c")
        sid = lax.axis_index("s")
        wid = sid * _NC + cid
        base = wid * ROWS

        pltpu.sync_copy(comment_hbm.at[pl.ds(base, ROWS)], idx_v)

        ones = jnp.ones((_LANES,), jnp.int32)
        zeros = jnp.zeros((_LANES,), jnp.int32)

        @pl.loop(0, ROWS)
        def _(rr):
            cnt = zeros
            for kk in range(LP // _LANES):
                v = idx_v[rr, pl.ds(kk * _LANES, _LANES)]
                cnt = cnt + jnp.where(v != 0, ones, zeros)
                g = ((v & (_KPACK - 1)) << 2) | (v >> 18)
                idxm_v[rr, pl.ds(kk * _LANES, _LANES)] = g
            len_sm[rr] = jnp.sum(cnt)

        def fire(b, r):
            pltpu.async_copy(
                table_hbm.at[idxm_v.at[r, pl.ds(0, C0)]],
                buf_v.at[b, pl.ds(0, C0)], sems[b])
            pltpu.async_copy(
                table_hbm.at[idxm_v.at[r, pl.ds(C0, C1)]],
                buf_v.at[b, pl.ds(C0, C1)], sems[b])

        for b in range(_NBUF):
            fire(b, b)

        @pl.loop(0, ROWS, step=_NBUF)
        def _(r0):
            for b in range(_NBUF):
                r = r0 + b
                pltpu.make_async_copy(
                    table_hbm.at[pl.ds(0, L)], buf_v.at[b], sems[b]).wait()

                length = len_sm[r]
                neg = jnp.full((_LANES,), -1e9, jnp.float32)
                n8 = length // 8

                def blk_body(i, carry):
                    a0, a1 = carry
                    for j in range(8):
                        p = i * 8 + j
                        a0 = jnp.maximum(a0, buf_v[b, p, pl.ds(0, _LANES)])
                        a1 = jnp.maximum(a1, buf_v[b, p, pl.ds(_LANES, _LANES)])
                    return a0, a1

                a0, a1 = lax.fori_loop(0, n8, blk_body, (neg, neg))

                def rem_body(p, carry):
                    a0, a1 = carry
                    a0 = jnp.maximum(a0, buf_v[b, p, pl.ds(0, _LANES)])
                    a1 = jnp.maximum(a1, buf_v[b, p, pl.ds(_LANES, _LANES)])
                    return a0, a1

                a0, a1 = lax.fori_loop(n8 * 8, length, rem_body, (a0, a1))

                pool_v[r, pl.ds(0, _LANES)] = a0
                pool_v[r, pl.ds(_LANES, _LANES)] = a1

                @pl.when(r + _NBUF < ROWS)
                def _():
                    fire(b, r + _NBUF)

        pltpu.sync_copy(pool_v, out_hbm.at[pl.ds(base, ROWS)])

    return k(comment_pad, emb_table)


_KPACK = 1 << 18


def _to_linear(tT):
    D, V = tT.shape
    R = 8192
    NQ = 4

    last_blk = (V - 1) // R

    def tr(x0, x1, x2, x3, o_ref):
        o_ref[...] = jnp.concatenate(
            [x0[...], x1[...], x2[...], x3[...]], axis=0).T

    return pl.pallas_call(
        tr,
        grid=(_KPACK // R,),
        in_specs=[
            pl.BlockSpec(
                (D, R),
                lambda i, q=q: (0, jnp.minimum(q * (_KPACK // R) + i,
                                               last_blk)))
            for q in range(NQ)
        ],
        out_specs=pl.BlockSpec((R, 4 * D), lambda i: (i, 0)),
        out_shape=jax.ShapeDtypeStruct((_KPACK, 4 * D), jnp.float32),
    )(tT, tT, tT, tT)


def _logits_mm(pooled, W, b):
    B, D = pooled.shape
    C = W.shape[1]

    def mm(x_ref, w_ref, b_ref, o_ref):
        o_ref[...] = (
            jnp.dot(x_ref[...], w_ref[...], preferred_element_type=jnp.float32)
            + b_ref[...]
        )

    return pl.pallas_call(
        mm,
        out_shape=jax.ShapeDtypeStruct((B, C), jnp.float32),
    )(pooled, W, b.reshape(1, C))


def kernel(comment, emb_table, W, b):
    comment = comment.astype(jnp.int32)
    lp = (comment.shape[1] + 15) // 16 * 16
    comment_pad = jnp.pad(comment, ((0, 0), (0, lp - comment.shape[1])))
    lin = _to_linear(emb_table.T)
    table_lin = lin.reshape(-1).reshape(4 * _KPACK, emb_table.shape[1])
    pooled = _sc_pool(comment_pad, table_lin)
    return _logits_mm(pooled, W, b)


# speedup vs baseline: 121.8126x; 1.0157x over previous
import dataclasses
import functools

import jax
import jax.numpy as jnp
from jax import lax
from jax.experimental import pallas as pl
from jax.experimental.pallas import tpu as pltpu
from jax.experimental.pallas import tpu_sc as plsc

_NC = 2
_NS = 16
_LANES = 16
_NBUF = 4


def _sc_pool(comment_pad, emb_table):
    B, LP = comment_pad.shape
    L = 200
    V, D = emb_table.shape
    NW = _NC * _NS
    ROWS = B // NW
    C0 = 128
    C1 = L - C0

    mesh = plsc.VectorSubcoreMesh(core_axis_name="c", subcore_axis_name="s")
    cp = pltpu.CompilerParams(needs_layout_passes=False,
                              use_tc_tiling_on_sc=False)

    @functools.partial(
        pl.kernel,
        out_type=jax.ShapeDtypeStruct((B, D), jnp.float32),
        mesh=mesh,
        compiler_params=cp,
        scratch_types=[
            pltpu.VMEM((ROWS, LP), jnp.int32),
            pltpu.VMEM((ROWS, LP), jnp.int32),
            pltpu.VMEM((_NBUF, L, D), jnp.float32),
            pltpu.VMEM((ROWS, D), jnp.float32),
            pltpu.SMEM((ROWS,), jnp.int32),
            pltpu.SemaphoreType.DMA,
            pltpu.SemaphoreType.DMA,
            pltpu.SemaphoreType.DMA,
            pltpu.SemaphoreType.DMA,
        ],
    )
    def k(comment_hbm, table_hbm, out_hbm, idx_v, idxm_v, buf_v, pool_v,
          len_sm, s0, s1, s2, s3):
        sems = (s0, s1, s2, s3)
        cid = lax.axis_index("c")
        sid = lax.axis_index("s")
        wid = sid * _NC + cid
        base = wid * ROWS

        pltpu.sync_copy(comment_hbm.at[pl.ds(base, ROWS)], idx_v)

        ones = jnp.ones((_LANES,), jnp.int32)
        zeros = jnp.zeros((_LANES,), jnp.int32)

        @pl.loop(0, ROWS)
        def _(rr):
            cnt = zeros
            for kk in range(LP // _LANES):
                v = idx_v[rr, pl.ds(kk * _LANES, _LANES)]
                cnt = cnt + jnp.where(v != 0, ones, zeros)
                g = ((v & (_KPACK - 1)) << 2) | (v >> 18)
                idxm_v[rr, pl.ds(kk * _LANES, _LANES)] = g
            len_sm[rr] = jnp.sum(cnt)

        def fire(b, r):
            pltpu.async_copy(
                table_hbm.at[idxm_v.at[r, pl.ds(0, C0)]],
                buf_v.at[b, pl.ds(0, C0)], sems[b])
            pltpu.async_copy(
                table_hbm.at[idxm_v.at[r, pl.ds(C0, C1)]],
                buf_v.at[b, pl.ds(C0, C1)], sems[b])

        for b in range(_NBUF):
            fire(b, b)

        @pl.loop(0, ROWS, step=_NBUF)
        def _(r0):
            for b in range(_NBUF):
                r = r0 + b
                pltpu.make_async_copy(
                    table_hbm.at[pl.ds(0, L)], buf_v.at[b], sems[b]).wait()

                length = len_sm[r]
                neg = jnp.full((_LANES,), -1e9, jnp.float32)
                n8 = length // 8

                def blk_body(i, carry):
                    a0, a1 = carry
                    for j in range(8):
                        p = i * 8 + j
                        a0 = jnp.maximum(a0, buf_v[b, p, pl.ds(0, _LANES)])
                        a1 = jnp.maximum(a1, buf_v[b, p, pl.ds(_LANES, _LANES)])
                    return a0, a1

                a0, a1 = lax.fori_loop(0, n8, blk_body, (neg, neg))

                def rem_body(p, carry):
                    a0, a1 = carry
                    a0 = jnp.maximum(a0, buf_v[b, p, pl.ds(0, _LANES)])
                    a1 = jnp.maximum(a1, buf_v[b, p, pl.ds(_LANES, _LANES)])
                    return a0, a1

                a0, a1 = lax.fori_loop(n8 * 8, length, rem_body, (a0, a1))

                pool_v[r, pl.ds(0, _LANES)] = a0
                pool_v[r, pl.ds(_LANES, _LANES)] = a1

                @pl.when(r + _NBUF < ROWS)
                def _():
                    fire(b, r + _NBUF)

        pltpu.sync_copy(pool_v, out_hbm.at[pl.ds(base, ROWS)])

    return k(comment_pad, emb_table)


_KPACK = 1 << 18


def _to_linear(tT):
    D, V = tT.shape
    R = 16384
    NQ = 4

    last_blk = (V - 1) // R

    def tr(x0, x1, x2, x3, o_ref):
        o_ref[...] = jnp.concatenate(
            [x0[...], x1[...], x2[...], x3[...]], axis=0).T

    return pl.pallas_call(
        tr,
        grid=(_KPACK // R,),
        in_specs=[
            pl.BlockSpec(
                (D, R),
                lambda i, q=q: (0, jnp.minimum(q * (_KPACK // R) + i,
                                               last_blk)))
            for q in range(NQ)
        ],
        out_specs=pl.BlockSpec((R, 4 * D), lambda i: (i, 0)),
        out_shape=jax.ShapeDtypeStruct((_KPACK, 4 * D), jnp.float32),
    )(tT, tT, tT, tT)


def _logits_mm(pooled, W, b):
    B, D = pooled.shape
    C = W.shape[1]

    def mm(x_ref, w_ref, b_ref, o_ref):
        o_ref[...] = (
            jnp.dot(x_ref[...], w_ref[...], preferred_element_type=jnp.float32)
            + b_ref[...]
        )

    return pl.pallas_call(
        mm,
        out_shape=jax.ShapeDtypeStruct((B, C), jnp.float32),
    )(pooled, W, b.reshape(1, C))


def kernel(comment, emb_table, W, b):
    comment = comment.astype(jnp.int32)
    lp = (comment.shape[1] + 15) // 16 * 16
    comment_pad = jnp.pad(comment, ((0, 0), (0, lp - comment.shape[1])))
    lin = _to_linear(emb_table.T)
    table_lin = lin.reshape(-1).reshape(4 * _KPACK, emb_table.shape[1])
    pooled = _sc_pool(comment_pad, table_lin)
    return _logits_mm(pooled, W, b)


# speedup vs baseline: 128.2868x; 1.0531x over previous
import dataclasses
import functools

import jax
import jax.numpy as jnp
from jax import lax
from jax.experimental import pallas as pl
from jax.experimental.pallas import tpu as pltpu
from jax.experimental.pallas import tpu_sc as plsc

_NC = 2
_NS = 16
_LANES = 16
_NBUF = 8


def _sc_pool(comment_pad, emb_table):
    B, LP = comment_pad.shape
    L = 200
    V, D = emb_table.shape
    NW = _NC * _NS
    ROWS = B // NW
    C0 = 128
    C1 = L - C0

    mesh = plsc.VectorSubcoreMesh(core_axis_name="c", subcore_axis_name="s")
    cp = pltpu.CompilerParams(needs_layout_passes=False,
                              use_tc_tiling_on_sc=False)

    @functools.partial(
        pl.kernel,
        out_type=jax.ShapeDtypeStruct((B, D), jnp.float32),
        mesh=mesh,
        compiler_params=cp,
        scratch_types=[
            pltpu.VMEM((ROWS, LP), jnp.int32),
            pltpu.VMEM((ROWS, LP), jnp.int32),
            pltpu.VMEM((_NBUF, L, D), jnp.float32),
            pltpu.VMEM((ROWS, D), jnp.float32),
            pltpu.SMEM((ROWS,), jnp.int32),
        ] + [pltpu.SemaphoreType.DMA] * _NBUF,
    )
    def k(comment_hbm, table_hbm, out_hbm, idx_v, idxm_v, buf_v, pool_v,
          len_sm, *sems):
        cid = lax.axis_index("c")
        sid = lax.axis_index("s")
        wid = sid * _NC + cid
        base = wid * ROWS

        pltpu.sync_copy(comment_hbm.at[pl.ds(base, ROWS)], idx_v)

        ones = jnp.ones((_LANES,), jnp.int32)
        zeros = jnp.zeros((_LANES,), jnp.int32)

        @pl.loop(0, ROWS)
        def _(rr):
            cnt = zeros
            for kk in range(LP // _LANES):
                v = idx_v[rr, pl.ds(kk * _LANES, _LANES)]
                cnt = cnt + jnp.where(v != 0, ones, zeros)
                g = ((v & (_KPACK - 1)) << 2) | (v >> 18)
                idxm_v[rr, pl.ds(kk * _LANES, _LANES)] = g
            len_sm[rr] = jnp.sum(cnt)

        def fire(b, r):
            pltpu.async_copy(
                table_hbm.at[idxm_v.at[r, pl.ds(0, C0)]],
                buf_v.at[b, pl.ds(0, C0)], sems[b])
            pltpu.async_copy(
                table_hbm.at[idxm_v.at[r, pl.ds(C0, C1)]],
                buf_v.at[b, pl.ds(C0, C1)], sems[b])

        for b in range(_NBUF):
            fire(b, b)

        @pl.loop(0, ROWS, step=_NBUF)
        def _(r0):
            for b in range(_NBUF):
                r = r0 + b
                pltpu.make_async_copy(
                    table_hbm.at[pl.ds(0, L)], buf_v.at[b], sems[b]).wait()

                length = len_sm[r]
                neg = jnp.full((_LANES,), -1e9, jnp.float32)
                n8 = length // 8

                def blk_body(i, carry):
                    a0, a1 = carry
                    for j in range(8):
                        p = i * 8 + j
                        a0 = jnp.maximum(a0, buf_v[b, p, pl.ds(0, _LANES)])
                        a1 = jnp.maximum(a1, buf_v[b, p, pl.ds(_LANES, _LANES)])
                    return a0, a1

                a0, a1 = lax.fori_loop(0, n8, blk_body, (neg, neg))

                def rem_body(p, carry):
                    a0, a1 = carry
                    a0 = jnp.maximum(a0, buf_v[b, p, pl.ds(0, _LANES)])
                    a1 = jnp.maximum(a1, buf_v[b, p, pl.ds(_LANES, _LANES)])
                    return a0, a1

                a0, a1 = lax.fori_loop(n8 * 8, length, rem_body, (a0, a1))

                pool_v[r, pl.ds(0, _LANES)] = a0
                pool_v[r, pl.ds(_LANES, _LANES)] = a1

                @pl.when(r + _NBUF < ROWS)
                def _():
                    fire(b, r + _NBUF)

        pltpu.sync_copy(pool_v, out_hbm.at[pl.ds(base, ROWS)])

    return k(comment_pad, emb_table)


_KPACK = 1 << 18


def _to_linear(tT):
    D, V = tT.shape
    R = 16384
    NQ = 4

    last_blk = (V - 1) // R

    def tr(x0, x1, x2, x3, o_ref):
        o_ref[...] = jnp.concatenate(
            [x0[...], x1[...], x2[...], x3[...]], axis=0).T

    return pl.pallas_call(
        tr,
        grid=(_KPACK // R,),
        in_specs=[
            pl.BlockSpec(
                (D, R),
                lambda i, q=q: (0, jnp.minimum(q * (_KPACK // R) + i,
                                               last_blk)))
            for q in range(NQ)
        ],
        out_specs=pl.BlockSpec((R, 4 * D), lambda i: (i, 0)),
        out_shape=jax.ShapeDtypeStruct((_KPACK, 4 * D), jnp.float32),
    )(tT, tT, tT, tT)


def _logits_mm(pooled, W, b):
    B, D = pooled.shape
    C = W.shape[1]

    def mm(x_ref, w_ref, b_ref, o_ref):
        o_ref[...] = (
            jnp.dot(x_ref[...], w_ref[...], preferred_element_type=jnp.float32)
            + b_ref[...]
        )

    return pl.pallas_call(
        mm,
        out_shape=jax.ShapeDtypeStruct((B, C), jnp.float32),
    )(pooled, W, b.reshape(1, C))


def kernel(comment, emb_table, W, b):
    comment = comment.astype(jnp.int32)
    lp = (comment.shape[1] + 15) // 16 * 16
    comment_pad = jnp.pad(comment, ((0, 0), (0, lp - comment.shape[1])))
    lin = _to_linear(emb_table.T)
    table_lin = lin.reshape(-1).reshape(4 * _KPACK, emb_table.shape[1])
    pooled = _sc_pool(comment_pad, table_lin)
    return _logits_mm(pooled, W, b)
